# Initial kernel scaffold; baseline (speedup 1.0000x reference)
import jax
import jax.numpy as jnp
from jax.experimental import pallas as pl


def kernel(hidden_states, router_w, gate_up_proj, down_proj, gate_w, up_w, down_w):
    raise NotImplementedError("write your pallas kernel here")


# speedup vs baseline: 1.0101x; 1.0101x over previous
import functools

import jax
import jax.numpy as jnp
from jax import lax
from jax.experimental import pallas as pl
from jax.experimental.pallas import tpu as pltpu
from jax.experimental.pallas import tpu_sc as plsc

E = 8
H = 768
F = 1024
T = 2048
BT = 128
C = 3072
NT = T // BT + E - 1
TA = 256

NC = 2
NS = 16
NW = NC * NS


def _router_body(x_ref, rw_ref, scaled_ref, eid_ref):
    x = x_ref[...]
    logits = lax.dot_general(x, rw_ref[...], (((1,), (1,)), ((), ())),
                             preferred_element_type=jnp.float32)
    score = jax.nn.sigmoid(jnp.max(logits, axis=1, keepdims=True))
    scaled_ref[...] = x * score
    eid_ref[...] = jnp.argmax(logits, axis=1).astype(jnp.int32)


def _router(hs, router_w):
    return pl.pallas_call(
        _router_body,
        grid=(T // TA,),
        in_specs=[
            pl.BlockSpec((TA, H), lambda i: (i, 0)),
            pl.BlockSpec((E, H), lambda i: (0, 0)),
        ],
        out_specs=[
            pl.BlockSpec((TA, H), lambda i: (i, 0)),
            pl.BlockSpec((TA,), lambda i: (i,)),
        ],
        out_shape=[
            jax.ShapeDtypeStruct((T, H), jnp.float32),
            jax.ShapeDtypeStruct((T,), jnp.int32),
        ],
    )(hs, router_w)


def _routing_meta(eid):
    oh = (eid[:, None] == jnp.arange(E, dtype=jnp.int32)[None, :]).astype(jnp.int32)
    counts = jnp.sum(oh, axis=0)
    nblk = (counts + BT - 1) // BT
    pstart = jnp.concatenate([jnp.zeros((1,), jnp.int32),
                              jnp.cumsum(nblk * BT)[:-1].astype(jnp.int32)])
    rank = jnp.take_along_axis(jnp.cumsum(oh, axis=0) - oh,
                               eid[:, None], axis=1)[:, 0]
    pos = pstart[eid] + rank
    iperm = jnp.zeros((C,), jnp.int32).at[pos].set(
        jnp.arange(T, dtype=jnp.int32))
    cn = jnp.cumsum(nblk)
    num_active = cn[-1]
    idx = jnp.minimum(jnp.arange(NT), num_active - 1)
    te = jnp.searchsorted(cn, idx, side="right").astype(jnp.int32)
    cne = (cn - nblk)[te]
    tb = (pstart[te] // BT + (idx - cne)).astype(jnp.int32)
    na = jnp.full((1,), num_active, jnp.int32)
    return pos, iperm, te, tb, na


@functools.lru_cache(maxsize=None)
def _make_sc_gather(n_rows, n_table):
    rpw = n_rows // NW
    mesh = plsc.VectorSubcoreMesh(core_axis_name="c", subcore_axis_name="s")

    @functools.partial(
        pl.kernel,
        out_type=jax.ShapeDtypeStruct((n_rows, H), jnp.float32),
        mesh=mesh,
        scratch_types=[
            pltpu.VMEM((rpw,), jnp.int32),
            pltpu.VMEM((rpw, H), jnp.float32),
            pltpu.SemaphoreType.DMA,
        ],
    )
    def k(idx_hbm, table_hbm, out_hbm, idx_v, rows_v, sem):
        wid = lax.axis_index("s") * NC + lax.axis_index("c")
        base = wid * rpw
        pltpu.sync_copy(idx_hbm.at[pl.ds(base, rpw)], idx_v)
        pltpu.async_copy(table_hbm.at[idx_v], rows_v, sem).wait()
        pltpu.sync_copy(rows_v, out_hbm.at[pl.ds(base, rpw)])

    return k


def _moe_body(te_ref, tb_ref, na_ref, x_ref, wgu_ref, wd_ref, o_ref):
    i = pl.program_id(0)

    @pl.when(i < na_ref[0])
    def _():
        x = x_ref[...]
        gu = jnp.dot(x, wgu_ref[0], preferred_element_type=jnp.float32)
        g = gu[:, :F]
        u = gu[:, F:]
        h = u * (g * jax.nn.sigmoid(g))
        o_ref[...] = jnp.dot(h, wd_ref[0], preferred_element_type=jnp.float32)


def _moe(sorted_x, gate_up_proj, down_proj, te, tb, na):
    grid_spec = pltpu.PrefetchScalarGridSpec(
        num_scalar_prefetch=3,
        grid=(NT,),
        in_specs=[
            pl.BlockSpec((BT, H), lambda i, te, tb, na: (tb[i], 0)),
            pl.BlockSpec((1, H, 2 * F), lambda i, te, tb, na: (te[i], 0, 0)),
            pl.BlockSpec((1, F, H), lambda i, te, tb, na: (te[i], 0, 0)),
        ],
        out_specs=pl.BlockSpec((BT, H), lambda i, te, tb, na: (tb[i], 0)),
    )
    return pl.pallas_call(
        _moe_body,
        grid_spec=grid_spec,
        out_shape=jax.ShapeDtypeStruct((C, H), jnp.float32),
    )(te, tb, na, sorted_x, gate_up_proj, down_proj)


def _shared_body(x_ref, gw_ref, uw_ref, dw_ref, r_ref, o_ref):
    x = x_ref[...]
    g = lax.dot_general(x, gw_ref[...], (((1,), (1,)), ((), ())),
                        preferred_element_type=jnp.float32)
    u = lax.dot_general(x, uw_ref[...], (((1,), (1,)), ((), ())),
                        preferred_element_type=jnp.float32)
    s = u * (g * jax.nn.sigmoid(g))
    o_ref[...] = lax.dot_general(s, dw_ref[...], (((1,), (1,)), ((), ())),
                                 preferred_element_type=jnp.float32) + r_ref[...]


def _shared(hs, gate_w, up_w, down_w, routed):
    return pl.pallas_call(
        _shared_body,
        grid=(T // TA,),
        in_specs=[
            pl.BlockSpec((TA, H), lambda i: (i, 0)),
            pl.BlockSpec((F, H), lambda i: (0, 0)),
            pl.BlockSpec((F, H), lambda i: (0, 0)),
            pl.BlockSpec((H, F), lambda i: (0, 0)),
            pl.BlockSpec((TA, H), lambda i: (i, 0)),
        ],
        out_specs=pl.BlockSpec((TA, H), lambda i: (i, 0)),
        out_shape=jax.ShapeDtypeStruct((T, H), jnp.float32),
    )(hs, gate_w, up_w, down_w, routed)


@jax.jit
def kernel(hidden_states, router_w, gate_up_proj, down_proj, gate_w, up_w, down_w):
    orig_shape = hidden_states.shape
    hs = hidden_states.reshape(-1, H)
    scaled, eid = _router(hs, router_w)
    pos, iperm, te, tb, na = _routing_meta(eid)
    sorted_x = _make_sc_gather(C, T)(iperm, scaled)
    sorted_out = _moe(sorted_x, gate_up_proj, down_proj, te, tb, na)
    routed = _make_sc_gather(T, C)(pos, sorted_out)
    out = _shared(hs, gate_w, up_w, down_w, routed)
    return out.reshape(orig_shape)


# speedup vs baseline: 1.3940x; 1.3800x over previous
import functools

import jax
import jax.numpy as jnp
from jax import lax
from jax.experimental import pallas as pl
from jax.experimental.pallas import tpu as pltpu
from jax.experimental.pallas import tpu_sc as plsc

E = 8
H = 768
F = 1024
T = 2048
BT = 128
C = 3072
NT = T // BT + E - 1
TA = 256

NC = 2
NS = 16
NW = NC * NS


def _router_body(x_ref, rw_ref, scaled_ref, eid_ref):
    x = x_ref[...]
    logits = lax.dot_general(x, rw_ref[...], (((1,), (1,)), ((), ())),
                             preferred_element_type=jnp.float32)
    score = jax.nn.sigmoid(jnp.max(logits, axis=1, keepdims=True))
    scaled_ref[...] = x * score
    eid_ref[...] = jnp.argmax(logits, axis=1).astype(jnp.int32)


def _router(hs, router_w):
    return pl.pallas_call(
        _router_body,
        grid=(T // TA,),
        in_specs=[
            pl.BlockSpec((TA, H), lambda i: (i, 0)),
            pl.BlockSpec((E, H), lambda i: (0, 0)),
        ],
        out_specs=[
            pl.BlockSpec((TA, H), lambda i: (i, 0)),
            pl.BlockSpec((TA,), lambda i: (i,)),
        ],
        out_shape=[
            jax.ShapeDtypeStruct((T, H), jnp.float32),
            jax.ShapeDtypeStruct((T,), jnp.int32),
        ],
    )(hs, router_w)


def _routing_meta(eid):
    oh = (eid[:, None] == jnp.arange(E, dtype=jnp.int32)[None, :]).astype(jnp.int32)
    counts = jnp.sum(oh, axis=0)
    nblk = (counts + BT - 1) // BT
    pstart = jnp.concatenate([jnp.zeros((1,), jnp.int32),
                              jnp.cumsum(nblk * BT)[:-1].astype(jnp.int32)])
    rank = jnp.take_along_axis(jnp.cumsum(oh, axis=0) - oh,
                               eid[:, None], axis=1)[:, 0]
    pos = pstart[eid] + rank
    iperm = (jnp.arange(C, dtype=jnp.int32) % T).at[pos].set(
        jnp.arange(T, dtype=jnp.int32))
    cn = jnp.cumsum(nblk)
    num_active = cn[-1]
    idx = jnp.minimum(jnp.arange(NT), num_active - 1)
    te = jnp.searchsorted(cn, idx, side="right").astype(jnp.int32)
    cne = (cn - nblk)[te]
    tb = (pstart[te] // BT + (idx - cne)).astype(jnp.int32)
    na = jnp.full((1,), num_active, jnp.int32)
    return pos, iperm, te, tb, na


@functools.lru_cache(maxsize=None)
def _make_sc_gather(n_rows, n_table):
    rpw = n_rows // NW
    mesh = plsc.VectorSubcoreMesh(core_axis_name="c", subcore_axis_name="s")

    @functools.partial(
        pl.kernel,
        out_type=jax.ShapeDtypeStruct((n_rows, H), jnp.float32),
        mesh=mesh,
        scratch_types=[
            pltpu.VMEM((rpw,), jnp.int32),
            pltpu.VMEM((rpw, H), jnp.float32),
            pltpu.SemaphoreType.DMA,
        ],
    )
    def k(idx_hbm, table_hbm, out_hbm, idx_v, rows_v, sem):
        wid = lax.axis_index("s") * NC + lax.axis_index("c")
        base = wid * rpw
        pltpu.sync_copy(idx_hbm.at[pl.ds(base, rpw)], idx_v)
        pltpu.async_copy(table_hbm.at[idx_v], rows_v, sem).wait()
        pltpu.sync_copy(rows_v, out_hbm.at[pl.ds(base, rpw)])

    return k


def _moe_body(te_ref, tb_ref, na_ref, x_ref, wgu_ref, wd_ref, o_ref):
    i = pl.program_id(0)

    @pl.when(i < na_ref[0])
    def _():
        x = x_ref[...]
        gu = jnp.dot(x, wgu_ref[0], preferred_element_type=jnp.float32)
        g = gu[:, :F]
        u = gu[:, F:]
        h = u * (g * jax.nn.sigmoid(g))
        o_ref[...] = jnp.dot(h, wd_ref[0], preferred_element_type=jnp.float32)


def _moe(sorted_x, gate_up_proj, down_proj, te, tb, na):
    grid_spec = pltpu.PrefetchScalarGridSpec(
        num_scalar_prefetch=3,
        grid=(NT,),
        in_specs=[
            pl.BlockSpec((BT, H), lambda i, te, tb, na: (tb[i], 0)),
            pl.BlockSpec((1, H, 2 * F), lambda i, te, tb, na: (te[i], 0, 0)),
            pl.BlockSpec((1, F, H), lambda i, te, tb, na: (te[i], 0, 0)),
        ],
        out_specs=pl.BlockSpec((BT, H), lambda i, te, tb, na: (tb[i], 0)),
    )
    return pl.pallas_call(
        _moe_body,
        grid_spec=grid_spec,
        out_shape=jax.ShapeDtypeStruct((C, H), jnp.float32),
    )(te, tb, na, sorted_x, gate_up_proj, down_proj)


def _shared_body(x_ref, gw_ref, uw_ref, dw_ref, r_ref, o_ref):
    x = x_ref[...]
    g = lax.dot_general(x, gw_ref[...], (((1,), (1,)), ((), ())),
                        preferred_element_type=jnp.float32)
    u = lax.dot_general(x, uw_ref[...], (((1,), (1,)), ((), ())),
                        preferred_element_type=jnp.float32)
    s = u * (g * jax.nn.sigmoid(g))
    o_ref[...] = lax.dot_general(s, dw_ref[...], (((1,), (1,)), ((), ())),
                                 preferred_element_type=jnp.float32) + r_ref[...]


def _shared(hs, gate_w, up_w, down_w, routed):
    return pl.pallas_call(
        _shared_body,
        grid=(T // TA,),
        in_specs=[
            pl.BlockSpec((TA, H), lambda i: (i, 0)),
            pl.BlockSpec((F, H), lambda i: (0, 0)),
            pl.BlockSpec((F, H), lambda i: (0, 0)),
            pl.BlockSpec((H, F), lambda i: (0, 0)),
            pl.BlockSpec((TA, H), lambda i: (i, 0)),
        ],
        out_specs=pl.BlockSpec((TA, H), lambda i: (i, 0)),
        out_shape=jax.ShapeDtypeStruct((T, H), jnp.float32),
    )(hs, gate_w, up_w, down_w, routed)


@jax.jit
def kernel(hidden_states, router_w, gate_up_proj, down_proj, gate_w, up_w, down_w):
    orig_shape = hidden_states.shape
    hs = hidden_states.reshape(-1, H)
    scaled, eid = _router(hs, router_w)
    pos, iperm, te, tb, na = _routing_meta(eid)
    sorted_x = _make_sc_gather(C, T)(iperm, scaled)
    sorted_out = _moe(sorted_x, gate_up_proj, down_proj, te, tb, na)
    routed = _make_sc_gather(T, C)(pos, sorted_out)
    out = _shared(hs, gate_w, up_w, down_w, routed)
    return out.reshape(orig_shape)


# speedup vs baseline: 1.5650x; 1.1227x over previous
import functools

import jax
import jax.numpy as jnp
from jax import lax
from jax.experimental import pallas as pl
from jax.experimental.pallas import tpu as pltpu
from jax.experimental.pallas import tpu_sc as plsc

E = 8
H = 768
F = 1024
T = 2048
BT = 128
C = 3072
NT = T // BT + E - 1
TA = 256

NC = 2
NS = 16
NW = NC * NS


def _router_body(x_ref, rw_ref, scaled_ref, eid_ref):
    x = x_ref[...]
    logits = lax.dot_general(x, rw_ref[...], (((1,), (1,)), ((), ())),
                             preferred_element_type=jnp.float32)
    score = jax.nn.sigmoid(jnp.max(logits, axis=1, keepdims=True))
    scaled_ref[...] = x * score
    eid_ref[...] = jnp.argmax(logits, axis=1).astype(jnp.int32)


def _router(hs, router_w):
    return pl.pallas_call(
        _router_body,
        grid=(T // TA,),
        in_specs=[
            pl.BlockSpec((TA, H), lambda i: (i, 0)),
            pl.BlockSpec((E, H), lambda i: (0, 0)),
        ],
        out_specs=[
            pl.BlockSpec((TA, H), lambda i: (i, 0)),
            pl.BlockSpec((TA,), lambda i: (i,)),
        ],
        out_shape=[
            jax.ShapeDtypeStruct((T, H), jnp.float32),
            jax.ShapeDtypeStruct((T,), jnp.int32),
        ],
    )(hs, router_w)


def _routing_meta(eid):
    oh = (eid[:, None] == jnp.arange(E, dtype=jnp.int32)[None, :]).astype(jnp.int32)
    counts = jnp.sum(oh, axis=0)
    nblk = (counts + BT - 1) // BT
    pstart = jnp.concatenate([jnp.zeros((1,), jnp.int32),
                              jnp.cumsum(nblk * BT)[:-1].astype(jnp.int32)])
    rank = jnp.sum(oh * (jnp.cumsum(oh, axis=0) - oh), axis=1)
    pos = jnp.sum(oh * pstart[None, :], axis=1) + rank
    cn = jnp.cumsum(nblk)
    num_active = cn[-1]
    idx = jnp.minimum(jnp.arange(NT), num_active - 1)
    te = jnp.searchsorted(cn, idx, side="right").astype(jnp.int32)
    cne = (cn - nblk)[te]
    tb = (pstart[te] // BT + (idx - cne)).astype(jnp.int32)
    na = jnp.full((1,), num_active, jnp.int32)
    return pos, te, tb, na


@functools.lru_cache(maxsize=None)
def _make_sc_scatter():
    rpw = T // NW
    mesh = plsc.VectorSubcoreMesh(core_axis_name="c", subcore_axis_name="s")

    @functools.partial(
        pl.kernel,
        out_type=jax.ShapeDtypeStruct((C, H), jnp.float32),
        mesh=mesh,
        scratch_types=[
            pltpu.VMEM((rpw,), jnp.int32),
            pltpu.VMEM((rpw, H), jnp.float32),
            pltpu.SemaphoreType.DMA,
        ],
    )
    def k(idx_hbm, rows_hbm, out_hbm, idx_v, rows_v, sem):
        wid = lax.axis_index("s") * NC + lax.axis_index("c")
        base = wid * rpw
        pltpu.sync_copy(idx_hbm.at[pl.ds(base, rpw)], idx_v)
        pltpu.sync_copy(rows_hbm.at[pl.ds(base, rpw)], rows_v)
        pltpu.async_copy(rows_v, out_hbm.at[idx_v], sem).wait()

    return k


@functools.lru_cache(maxsize=None)
def _make_sc_gather_add():
    rpw = T // NW
    nch = H // 16
    mesh = plsc.VectorSubcoreMesh(core_axis_name="c", subcore_axis_name="s")

    @functools.partial(
        pl.kernel,
        out_type=jax.ShapeDtypeStruct((T, H), jnp.float32),
        mesh=mesh,
        scratch_types=[
            pltpu.VMEM((rpw,), jnp.int32),
            pltpu.VMEM((rpw, H), jnp.float32),
            pltpu.VMEM((rpw, H), jnp.float32),
            pltpu.SemaphoreType.DMA,
        ],
    )
    def k(idx_hbm, table_hbm, shared_hbm, out_hbm, idx_v, rows_v, sh_v, sem):
        wid = lax.axis_index("s") * NC + lax.axis_index("c")
        base = wid * rpw
        pltpu.sync_copy(idx_hbm.at[pl.ds(base, rpw)], idx_v)
        cp = pltpu.async_copy(table_hbm.at[idx_v], rows_v, sem)
        pltpu.sync_copy(shared_hbm.at[pl.ds(base, rpw)], sh_v)
        cp.wait()

        def row(r, _):
            for c in range(nch):
                s = pl.ds(c * 16, 16)
                rows_v[r, s] = rows_v[r, s] + sh_v[r, s]
            return 0

        lax.fori_loop(0, rpw, row, 0)
        pltpu.sync_copy(rows_v, out_hbm.at[pl.ds(base, rpw)])

    return k


def _moe_body(te_ref, tb_ref, na_ref, x_ref, wgu_ref, wd_ref, o_ref):
    i = pl.program_id(0)

    @pl.when(i < na_ref[0])
    def _():
        x = x_ref[...]
        gu = jnp.dot(x, wgu_ref[0], preferred_element_type=jnp.float32,
                     precision=lax.Precision.DEFAULT)
        g = gu[:, :F]
        u = gu[:, F:]
        h = u * (g * jax.nn.sigmoid(g))
        o_ref[...] = jnp.dot(h, wd_ref[0], preferred_element_type=jnp.float32,
                             precision=lax.Precision.DEFAULT)


def _moe(sorted_x, gate_up_proj, down_proj, te, tb, na):
    grid_spec = pltpu.PrefetchScalarGridSpec(
        num_scalar_prefetch=3,
        grid=(NT,),
        in_specs=[
            pl.BlockSpec((BT, H), lambda i, te, tb, na: (tb[i], 0)),
            pl.BlockSpec((1, H, 2 * F), lambda i, te, tb, na: (te[i], 0, 0)),
            pl.BlockSpec((1, F, H), lambda i, te, tb, na: (te[i], 0, 0)),
        ],
        out_specs=pl.BlockSpec((BT, H), lambda i, te, tb, na: (tb[i], 0)),
    )
    return pl.pallas_call(
        _moe_body,
        grid_spec=grid_spec,
        out_shape=jax.ShapeDtypeStruct((C, H), jnp.float32),
    )(te, tb, na, sorted_x, gate_up_proj, down_proj)


def _shared_body(x_ref, gw_ref, uw_ref, dw_ref, o_ref):
    x = x_ref[...]
    cd = (((1,), (1,)), ((), ()))
    g = lax.dot_general(x, gw_ref[...], cd, preferred_element_type=jnp.float32,
                        precision=lax.Precision.DEFAULT)
    u = lax.dot_general(x, uw_ref[...], cd, preferred_element_type=jnp.float32,
                        precision=lax.Precision.DEFAULT)
    s = u * (g * jax.nn.sigmoid(g))
    o_ref[...] = lax.dot_general(s, dw_ref[...], cd,
                                 preferred_element_type=jnp.float32,
                                 precision=lax.Precision.DEFAULT)


def _shared(hs, gate_w, up_w, down_w):
    return pl.pallas_call(
        _shared_body,
        grid=(T // TA,),
        in_specs=[
            pl.BlockSpec((TA, H), lambda i: (i, 0)),
            pl.BlockSpec((F, H), lambda i: (0, 0)),
            pl.BlockSpec((F, H), lambda i: (0, 0)),
            pl.BlockSpec((H, F), lambda i: (0, 0)),
        ],
        out_specs=pl.BlockSpec((TA, H), lambda i: (i, 0)),
        out_shape=jax.ShapeDtypeStruct((T, H), jnp.float32),
    )(hs, gate_w, up_w, down_w)


@jax.jit
def kernel(hidden_states, router_w, gate_up_proj, down_proj, gate_w, up_w, down_w):
    orig_shape = hidden_states.shape
    hs = hidden_states.reshape(-1, H)
    scaled, eid = _router(hs, router_w)
    pos, te, tb, na = _routing_meta(eid)
    sorted_x = _make_sc_scatter()(pos, scaled)
    shared = _shared(hs, gate_w, up_w, down_w)
    sorted_out = _moe(sorted_x, gate_up_proj, down_proj, te, tb, na)
    out = _make_sc_gather_add()(pos, sorted_out, shared)
    return out.reshape(orig_shape)


# speedup vs baseline: 1.7513x; 1.1191x over previous
import functools

import jax
import jax.numpy as jnp
from jax import lax
from jax.experimental import pallas as pl
from jax.experimental.pallas import tpu as pltpu
from jax.experimental.pallas import tpu_sc as plsc

E = 8
H = 768
F = 1024
T = 2048
BT = 512
C = 6144
NT = T // BT + E - 1
TA = 256

NC = 2
NS = 16
NW = NC * NS


def _router_body(x_ref, rw_ref, scaled_ref, eid_ref):
    x = x_ref[...]
    logits = lax.dot_general(x, rw_ref[...], (((1,), (1,)), ((), ())),
                             preferred_element_type=jnp.float32)
    score = jax.nn.sigmoid(jnp.max(logits, axis=1, keepdims=True))
    scaled_ref[...] = x * score
    eid_ref[...] = jnp.argmax(logits, axis=1).astype(jnp.int32)


def _router(hs, router_w):
    return pl.pallas_call(
        _router_body,
        grid=(T // TA,),
        in_specs=[
            pl.BlockSpec((TA, H), lambda i: (i, 0)),
            pl.BlockSpec((E, H), lambda i: (0, 0)),
        ],
        out_specs=[
            pl.BlockSpec((TA, H), lambda i: (i, 0)),
            pl.BlockSpec((TA,), lambda i: (i,)),
        ],
        out_shape=[
            jax.ShapeDtypeStruct((T, H), jnp.float32),
            jax.ShapeDtypeStruct((T,), jnp.int32),
        ],
    )(hs, router_w)


def _routing_meta(eid):
    oh = (eid[:, None] == jnp.arange(E, dtype=jnp.int32)[None, :]).astype(jnp.int32)
    counts = jnp.sum(oh, axis=0)
    nblk = (counts + BT - 1) // BT
    pstart = jnp.concatenate([jnp.zeros((1,), jnp.int32),
                              jnp.cumsum(nblk * BT)[:-1].astype(jnp.int32)])
    rank = jnp.sum(oh * (jnp.cumsum(oh, axis=0) - oh), axis=1)
    pos = jnp.sum(oh * pstart[None, :], axis=1) + rank
    cn = jnp.cumsum(nblk)
    num_active = cn[-1]
    idx = jnp.minimum(jnp.arange(NT), num_active - 1)
    te = jnp.searchsorted(cn, idx, side="right").astype(jnp.int32)
    cne = (cn - nblk)[te]
    tb = (pstart[te] // BT + (idx - cne)).astype(jnp.int32)
    na = jnp.full((1,), num_active, jnp.int32)
    return pos, te, tb, na


@functools.lru_cache(maxsize=None)
def _make_sc_scatter():
    rpw = T // NW
    mesh = plsc.VectorSubcoreMesh(core_axis_name="c", subcore_axis_name="s")

    @functools.partial(
        pl.kernel,
        out_type=jax.ShapeDtypeStruct((C, H), jnp.float32),
        mesh=mesh,
        scratch_types=[
            pltpu.VMEM((rpw,), jnp.int32),
            pltpu.VMEM((rpw, H), jnp.float32),
            pltpu.SemaphoreType.DMA,
        ],
    )
    def k(idx_hbm, rows_hbm, out_hbm, idx_v, rows_v, sem):
        wid = lax.axis_index("s") * NC + lax.axis_index("c")
        base = wid * rpw
        pltpu.sync_copy(idx_hbm.at[pl.ds(base, rpw)], idx_v)
        pltpu.sync_copy(rows_hbm.at[pl.ds(base, rpw)], rows_v)
        pltpu.async_copy(rows_v, out_hbm.at[idx_v], sem).wait()

    return k


@functools.lru_cache(maxsize=None)
def _make_sc_gather_add():
    rpw = T // NW
    nch = H // 16
    mesh = plsc.VectorSubcoreMesh(core_axis_name="c", subcore_axis_name="s")

    @functools.partial(
        pl.kernel,
        out_type=jax.ShapeDtypeStruct((T, H), jnp.float32),
        mesh=mesh,
        scratch_types=[
            pltpu.VMEM((rpw,), jnp.int32),
            pltpu.VMEM((rpw, H), jnp.float32),
            pltpu.VMEM((rpw, H), jnp.float32),
            pltpu.SemaphoreType.DMA,
        ],
    )
    def k(idx_hbm, table_hbm, shared_hbm, out_hbm, idx_v, rows_v, sh_v, sem):
        wid = lax.axis_index("s") * NC + lax.axis_index("c")
        base = wid * rpw
        pltpu.sync_copy(idx_hbm.at[pl.ds(base, rpw)], idx_v)
        cp = pltpu.async_copy(table_hbm.at[idx_v], rows_v, sem)
        pltpu.sync_copy(shared_hbm.at[pl.ds(base, rpw)], sh_v)
        cp.wait()

        def row(r, _):
            for c in range(nch):
                s = pl.ds(c * 16, 16)
                rows_v[r, s] = rows_v[r, s] + sh_v[r, s]
            return 0

        lax.fori_loop(0, rpw, row, 0)
        pltpu.sync_copy(rows_v, out_hbm.at[pl.ds(base, rpw)])

    return k


def _moe_body(te_ref, tb_ref, na_ref, x_ref, wgu_ref, wd_ref, o_ref):
    i = pl.program_id(0)

    @pl.when(i < na_ref[0])
    def _():
        x = x_ref[...]
        gu = jnp.dot(x, wgu_ref[0], preferred_element_type=jnp.float32,
                     precision=lax.Precision.DEFAULT)
        g = gu[:, :F]
        u = gu[:, F:]
        h = u * (g * jax.nn.sigmoid(g))
        o_ref[...] = jnp.dot(h, wd_ref[0], preferred_element_type=jnp.float32,
                             precision=lax.Precision.DEFAULT)


def _moe(sorted_x, gate_up_proj, down_proj, te, tb, na):
    grid_spec = pltpu.PrefetchScalarGridSpec(
        num_scalar_prefetch=3,
        grid=(NT,),
        in_specs=[
            pl.BlockSpec((BT, H), lambda i, te, tb, na: (tb[i], 0)),
            pl.BlockSpec((1, H, 2 * F), lambda i, te, tb, na: (te[i], 0, 0)),
            pl.BlockSpec((1, F, H), lambda i, te, tb, na: (te[i], 0, 0)),
        ],
        out_specs=pl.BlockSpec((BT, H), lambda i, te, tb, na: (tb[i], 0)),
    )
    return pl.pallas_call(
        _moe_body,
        grid_spec=grid_spec,
        out_shape=jax.ShapeDtypeStruct((C, H), jnp.float32),
    )(te, tb, na, sorted_x, gate_up_proj, down_proj)


def _shared_body(x_ref, gw_ref, uw_ref, dw_ref, o_ref):
    x = x_ref[...]
    cd = (((1,), (1,)), ((), ()))
    g = lax.dot_general(x, gw_ref[...], cd, preferred_element_type=jnp.float32,
                        precision=lax.Precision.DEFAULT)
    u = lax.dot_general(x, uw_ref[...], cd, preferred_element_type=jnp.float32,
                        precision=lax.Precision.DEFAULT)
    s = u * (g * jax.nn.sigmoid(g))
    o_ref[...] = lax.dot_general(s, dw_ref[...], cd,
                                 preferred_element_type=jnp.float32,
                                 precision=lax.Precision.DEFAULT)


def _shared(hs, gate_w, up_w, down_w):
    return pl.pallas_call(
        _shared_body,
        grid=(T // TA,),
        in_specs=[
            pl.BlockSpec((TA, H), lambda i: (i, 0)),
            pl.BlockSpec((F, H), lambda i: (0, 0)),
            pl.BlockSpec((F, H), lambda i: (0, 0)),
            pl.BlockSpec((H, F), lambda i: (0, 0)),
        ],
        out_specs=pl.BlockSpec((TA, H), lambda i: (i, 0)),
        out_shape=jax.ShapeDtypeStruct((T, H), jnp.float32),
    )(hs, gate_w, up_w, down_w)


@jax.jit
def kernel(hidden_states, router_w, gate_up_proj, down_proj, gate_w, up_w, down_w):
    orig_shape = hidden_states.shape
    hs = hidden_states.reshape(-1, H)
    scaled, eid = _router(hs, router_w)
    pos, te, tb, na = _routing_meta(eid)
    sorted_x = _make_sc_scatter()(pos, scaled)
    shared = _shared(hs, gate_w, up_w, down_w)
    sorted_out = _moe(sorted_x, gate_up_proj, down_proj, te, tb, na)
    out = _make_sc_gather_add()(pos, sorted_out, shared)
    return out.reshape(orig_shape)
